# baseline (speedup 1.0000x reference)
import jax
import jax.numpy as jnp
from jax.experimental import pallas as pl
from jax.experimental.pallas import tpu as pltpu

_NSTAGES = 4
_K = 1024
_D = 32
_COMMIT_W = 0.25
_TB = 2048
_NWAY = 2


def _row_sum32(xx):
    v = ((xx[:, 0:8] + xx[:, 8:16]) + xx[:, 16:24]) + xx[:, 24:32]
    for sz in (4, 2, 1):
        v = v[:, :sz] + v[:, sz:2 * sz]
    return v


def _rvq_block(z_ref, cb_ref, q_ref, idx_ref, comm_ref, csum_ref):
    @pl.when(pl.program_id(0) == 0)
    def _():
        for s in range(_NSTAGES):
            c = cb_ref[s]
            csum_ref[s, :] = _row_sum32(c * c)[:, 0]

    _H = _TB // _NWAY
    iota = jax.lax.broadcasted_iota(jnp.int32, (_H, _K), 1)
    resid = [z_ref[pl.ds(h * _H, _H), :] for h in range(_NWAY)]
    acc = [jnp.zeros_like(r) for r in resid]
    comm = [jnp.float32(0.0)] * _NWAY
    for s in range(_NSTAGES):
        cb = cb_ref[s]
        csum = csum_ref[s]
        c1 = cb.astype(jnp.bfloat16)
        r1 = cb - c1.astype(jnp.float32)
        c2 = r1.astype(jnp.bfloat16)
        r2 = r1 - c2.astype(jnp.float32)
        c3 = r2.astype(jnp.bfloat16)
        r3 = r2 - c3.astype(jnp.float32)
        c4 = r3.astype(jnp.bfloat16)
        for h in range(_NWAY):
            r = resid[h]
            rsum = _row_sum32(r * r)
            mm2 = jax.lax.dot_general(
                r * (-2.0), cb, (((1,), (1,)), ((), ())),
                preferred_element_type=jnp.float32)
            dist = (rsum + csum[None, :]) + mm2
            m = jnp.min(dist, axis=1, keepdims=True)
            idx = jnp.min(jnp.where(dist <= m, iota, _K), axis=1)
            onehot = (iota == idx[:, None]).astype(jnp.bfloat16)
            dn = (((1,), (0,)), ((), ()))
            q = jax.lax.dot_general(
                onehot, c1, dn, preferred_element_type=jnp.float32)
            q = q + jax.lax.dot_general(
                onehot, c2, dn, preferred_element_type=jnp.float32)
            q = q + jax.lax.dot_general(
                onehot, c3, dn, preferred_element_type=jnp.float32)
            q = q + jax.lax.dot_general(
                onehot, c4, dn, preferred_element_type=jnp.float32)
            diff = q - r
            comm[h] = comm[h] + jnp.sum(diff * diff)
            acc[h] = acc[h] + (r + diff)
            resid[h] = r - q
            idx_ref[0, 0, pl.ds(s * _TB + h * _H, _H)] = idx
    q_ref[...] = jnp.concatenate(acc, axis=0)
    comm_ref[0, 0, :] = jnp.broadcast_to(sum(comm), (128,))


def kernel(z, codebooks):
    B, T, D = z.shape
    S, K, _ = codebooks.shape
    ntok = B * T
    nb = ntok // _TB
    zf = z.reshape(ntok, D)
    qf, idxs, commp = pl.pallas_call(
        _rvq_block,
        grid=(nb,),
        in_specs=[
            pl.BlockSpec((_TB, D), lambda i: (i, 0)),
            pl.BlockSpec((S, K, D), lambda i: (0, 0, 0)),
        ],
        out_specs=[
            pl.BlockSpec((_TB, D), lambda i: (i, 0)),
            pl.BlockSpec((1, 1, S * _TB), lambda i: (i, 0, 0)),
            pl.BlockSpec((1, 1, 128), lambda i: (i, 0, 0)),
        ],
        out_shape=[
            jax.ShapeDtypeStruct((ntok, D), jnp.float32),
            jax.ShapeDtypeStruct((nb, 1, S * _TB), jnp.int32),
            jax.ShapeDtypeStruct((nb, 1, 128), jnp.float32),
        ],
        scratch_shapes=[pltpu.VMEM((S, K), jnp.float32)],
        compiler_params=pltpu.CompilerParams(
            dimension_semantics=("arbitrary",)),
    )(zf, codebooks)
    quantized = qf.reshape(B, T, D)
    indices = idxs.reshape(nb, S, _TB).transpose(1, 0, 2).reshape(S, B, T)
    commitment_loss = (_COMMIT_W / (B * T * D)) * jnp.sum(commp[:, 0, 0])
    return quantized, commitment_loss, indices


# speedup vs baseline: 2.3135x; 2.3135x over previous
import jax
import jax.numpy as jnp
from jax.experimental import pallas as pl
from jax.experimental.pallas import tpu as pltpu

_NSTAGES = 4
_K = 1024
_D = 32
_COMMIT_W = 0.25
_TB = 2048
_NWAY = 2


def _row_sum32(xx):
    v = ((xx[:, 0:8] + xx[:, 8:16]) + xx[:, 16:24]) + xx[:, 24:32]
    for sz in (4, 2, 1):
        v = v[:, :sz] + v[:, sz:2 * sz]
    return v


def _rvq_block(z_ref, cb_ref, q_ref, idx_ref, comm_ref, csum_ref):
    @pl.when(pl.program_id(0) == 0)
    def _():
        for s in range(_NSTAGES):
            c = cb_ref[s]
            csum_ref[s, :] = _row_sum32(c * c)[:, 0]

    _H = _TB // _NWAY
    iota = jax.lax.broadcasted_iota(jnp.int32, (_H, _K), 1)
    riota = (jnp.int32(_K) - iota).astype(jnp.float32)
    resid = [z_ref[pl.ds(h * _H, _H), :] for h in range(_NWAY)]
    acc = [jnp.zeros_like(r) for r in resid]
    comm = [jnp.float32(0.0)] * _NWAY
    for s in range(_NSTAGES):
        cb = cb_ref[s]
        csum = csum_ref[s]
        c1 = cb.astype(jnp.bfloat16)
        r1 = cb - c1.astype(jnp.float32)
        c2 = r1.astype(jnp.bfloat16)
        r2 = r1 - c2.astype(jnp.float32)
        c3 = r2.astype(jnp.bfloat16)
        r3 = r2 - c3.astype(jnp.float32)
        c4 = r3.astype(jnp.bfloat16)
        cbq = jnp.concatenate([c1, c2, c3, c4], axis=1)
        for h in range(_NWAY):
            r = resid[h]
            rsum = _row_sum32(r * r)
            mm2 = jax.lax.dot_general(
                r * (-2.0), cb, (((1,), (1,)), ((), ())),
                preferred_element_type=jnp.float32)
            dist = (rsum + csum[None, :]) + mm2
            m = jnp.min(dist, axis=1, keepdims=True)
            mx = jnp.max(jnp.where(dist <= m, riota, 0.0), axis=1)
            idx = (jnp.float32(_K) - mx).astype(jnp.int32)
            onehot = (iota == idx[:, None]).astype(jnp.bfloat16)
            qcat = jax.lax.dot_general(
                onehot, cbq, (((1,), (0,)), ((), ())),
                preferred_element_type=jnp.float32)
            q = ((qcat[:, 0:_D] + qcat[:, _D:2 * _D])
                 + qcat[:, 2 * _D:3 * _D]) + qcat[:, 3 * _D:4 * _D]
            diff = q - r
            comm[h] = comm[h] + jnp.sum(diff * diff)
            acc[h] = acc[h] + (r + diff)
            resid[h] = r - q
            idx_ref[0, 0, pl.ds(s * _TB + h * _H, _H)] = idx
    q_ref[...] = jnp.concatenate(acc, axis=0)
    comm_ref[0, 0, :] = jnp.broadcast_to(sum(comm), (128,))


def kernel(z, codebooks):
    B, T, D = z.shape
    S, K, _ = codebooks.shape
    ntok = B * T
    nb = ntok // _TB
    zf = z.reshape(ntok, D)
    qf, idxs, commp = pl.pallas_call(
        _rvq_block,
        grid=(nb,),
        in_specs=[
            pl.BlockSpec((_TB, D), lambda i: (i, 0)),
            pl.BlockSpec((S, K, D), lambda i: (0, 0, 0)),
        ],
        out_specs=[
            pl.BlockSpec((_TB, D), lambda i: (i, 0)),
            pl.BlockSpec((1, 1, S * _TB), lambda i: (i, 0, 0)),
            pl.BlockSpec((1, 1, 128), lambda i: (i, 0, 0)),
        ],
        out_shape=[
            jax.ShapeDtypeStruct((ntok, D), jnp.float32),
            jax.ShapeDtypeStruct((nb, 1, S * _TB), jnp.int32),
            jax.ShapeDtypeStruct((nb, 1, 128), jnp.float32),
        ],
        scratch_shapes=[pltpu.VMEM((S, K), jnp.float32)],
        compiler_params=pltpu.CompilerParams(
            dimension_semantics=("arbitrary",)),
    )(zf, codebooks)
    quantized = qf.reshape(B, T, D)
    indices = idxs.reshape(nb, S, _TB).transpose(1, 0, 2).reshape(S, B, T)
    commitment_loss = (_COMMIT_W / (B * T * D)) * jnp.sum(commp[:, 0, 0])
    return quantized, commitment_loss, indices


# speedup vs baseline: 2.4099x; 1.0417x over previous
import jax
import jax.numpy as jnp
from jax.experimental import pallas as pl
from jax.experimental.pallas import tpu as pltpu

_NSTAGES = 4
_K = 1024
_D = 32
_COMMIT_W = 0.25
_TB = 2048
_NWAY = 4


def _row_sum32(xx):
    v = ((xx[:, 0:8] + xx[:, 8:16]) + xx[:, 16:24]) + xx[:, 24:32]
    for sz in (4, 2, 1):
        v = v[:, :sz] + v[:, sz:2 * sz]
    return v


def _rvq_block(z_ref, cb_ref, q_ref, idx_ref, comm_ref, csum_ref):
    @pl.when(pl.program_id(0) == 0)
    def _():
        for s in range(_NSTAGES):
            c = cb_ref[s]
            csum_ref[s, :] = _row_sum32(c * c)[:, 0]

    _H = _TB // _NWAY
    iota = jax.lax.broadcasted_iota(jnp.int32, (_H, _K), 1)
    riota = (jnp.int32(_K) - iota).astype(jnp.float32)
    resid = [z_ref[pl.ds(h * _H, _H), :] for h in range(_NWAY)]
    acc = [jnp.zeros_like(r) for r in resid]
    comm = [jnp.float32(0.0)] * _NWAY
    for s in range(_NSTAGES):
        cb = cb_ref[s]
        csum = csum_ref[s]
        c1 = cb.astype(jnp.bfloat16)
        r1 = cb - c1.astype(jnp.float32)
        c2 = r1.astype(jnp.bfloat16)
        r2 = r1 - c2.astype(jnp.float32)
        c3 = r2.astype(jnp.bfloat16)
        r3 = r2 - c3.astype(jnp.float32)
        c4 = r3.astype(jnp.bfloat16)
        cbq = jnp.concatenate([c1, c2, c3, c4], axis=1)
        for h in range(_NWAY):
            r = resid[h]
            rsum = _row_sum32(r * r)
            mm2 = jax.lax.dot_general(
                r * (-2.0), cb, (((1,), (1,)), ((), ())),
                preferred_element_type=jnp.float32)
            dist = (rsum + csum[None, :]) + mm2
            m = jnp.min(dist, axis=1, keepdims=True)
            mx = jnp.max(jnp.where(dist <= m, riota, 0.0), axis=1)
            idx = (jnp.float32(_K) - mx).astype(jnp.int32)
            onehot = (iota == idx[:, None]).astype(jnp.bfloat16)
            qcat = jax.lax.dot_general(
                onehot, cbq, (((1,), (0,)), ((), ())),
                preferred_element_type=jnp.float32)
            q = ((qcat[:, 0:_D] + qcat[:, _D:2 * _D])
                 + qcat[:, 2 * _D:3 * _D]) + qcat[:, 3 * _D:4 * _D]
            diff = q - r
            comm[h] = comm[h] + jnp.sum(diff * diff)
            acc[h] = acc[h] + (r + diff)
            resid[h] = r - q
            idx_ref[0, 0, pl.ds(s * _TB + h * _H, _H)] = idx
    q_ref[...] = jnp.concatenate(acc, axis=0)
    comm_ref[0, 0, :] = jnp.broadcast_to(sum(comm), (128,))


def kernel(z, codebooks):
    B, T, D = z.shape
    S, K, _ = codebooks.shape
    ntok = B * T
    nb = ntok // _TB
    zf = z.reshape(ntok, D)
    qf, idxs, commp = pl.pallas_call(
        _rvq_block,
        grid=(nb,),
        in_specs=[
            pl.BlockSpec((_TB, D), lambda i: (i, 0)),
            pl.BlockSpec((S, K, D), lambda i: (0, 0, 0)),
        ],
        out_specs=[
            pl.BlockSpec((_TB, D), lambda i: (i, 0)),
            pl.BlockSpec((1, 1, S * _TB), lambda i: (i, 0, 0)),
            pl.BlockSpec((1, 1, 128), lambda i: (i, 0, 0)),
        ],
        out_shape=[
            jax.ShapeDtypeStruct((ntok, D), jnp.float32),
            jax.ShapeDtypeStruct((nb, 1, S * _TB), jnp.int32),
            jax.ShapeDtypeStruct((nb, 1, 128), jnp.float32),
        ],
        scratch_shapes=[pltpu.VMEM((S, K), jnp.float32)],
        compiler_params=pltpu.CompilerParams(
            dimension_semantics=("arbitrary",)),
    )(zf, codebooks)
    quantized = qf.reshape(B, T, D)
    indices = idxs.reshape(nb, S, _TB).transpose(1, 0, 2).reshape(S, B, T)
    commitment_loss = (_COMMIT_W / (B * T * D)) * jnp.sum(commp[:, 0, 0])
    return quantized, commitment_loss, indices
